# baseline (speedup 1.0000x reference)
import functools

import jax
import jax.numpy as jnp
from jax import lax
from jax.experimental import pallas as pl
from jax.experimental.pallas import tpu as pltpu
from jax.experimental.pallas import tpu_sc as plsc

N, C, D = 1024, 2048, 64
V_GENE, V_MOD, V_EXPR = 1000000, 100, 16
T = N * C

NUM_CORES = 2
NUM_SUBCORES = 16
NW = NUM_CORES * NUM_SUBCORES
TOK_W = T // NW
CB = 128
K = 4
CHUNK = K * CB
NB = TOK_W // CHUNK
ROWS_W = TOK_W // CB

V_MOD_PAD = 128
BT = 2048


def _sc_gather(ids2d, table):
    mesh = plsc.VectorSubcoreMesh(core_axis_name="c", subcore_axis_name="s")

    @functools.partial(
        pl.kernel,
        mesh=mesh,
        compiler_params=pltpu.CompilerParams(use_tc_tiling_on_sc=False),
        out_type=jax.ShapeDtypeStruct((T // 2, 128), jnp.float32),
        scratch_types=[
            pltpu.VMEM((2, K, CB), jnp.int32),
            pltpu.VMEM((2, CHUNK, D), jnp.float32),
            pltpu.SemaphoreType.DMA,
            pltpu.SemaphoreType.DMA,
        ],
    )
    def body(ids_hbm, table_hbm, out_hbm, idxv, rows, sem0, sem1):
        wid = lax.axis_index("s") * NUM_CORES + lax.axis_index("c")
        row0 = wid * ROWS_W
        tok0 = (wid % (NW // 2)) * TOK_W

        def load(j, b):
            pltpu.sync_copy(ids_hbm.at[pl.ds(row0 + j * K, K)], idxv.at[b])
            for k in range(K):
                for m in range(CB // 16):
                    g = idxv[b, k, pl.ds(m * 16, 16)]
                    idxv[b, k, pl.ds(m * 16, 16)] = (
                        ((g >> 11) << 11) + ((g & 1023) << 1)
                        + ((g >> 10) & 1))

        def fire(b, sem):
            for k in range(K):
                pltpu.async_copy(table_hbm.at[idxv.at[b, k]],
                                 rows.at[b].at[pl.ds(k * CB, CB)], sem)

        def drain(b, sem):
            for k in range(K):
                pltpu.make_async_copy(table_hbm.at[pl.ds(0, CB)],
                                      rows.at[b].at[pl.ds(k * CB, CB)],
                                      sem).wait()

        def store(j, b):
            @pl.when(wid < NW // 2)
            def _():
                pltpu.sync_copy(
                    rows.at[b],
                    out_hbm.at[pl.ds(tok0 + j * CHUNK, CHUNK), pl.ds(0, D)])

            @pl.when(wid >= NW // 2)
            def _():
                pltpu.sync_copy(
                    rows.at[b],
                    out_hbm.at[pl.ds(tok0 + j * CHUNK, CHUNK), pl.ds(D, D)])

        load(0, 0)
        fire(0, sem0)

        def pair(jj, carry):
            b0 = 2 * jj
            b1 = b0 + 1
            load(b1, 1)
            fire(1, sem1)
            drain(0, sem0)
            store(b0, 0)
            nxt = b1 + 1

            @pl.when(nxt < NB)
            def _():
                load(nxt, 0)
                fire(0, sem0)

            drain(1, sem1)
            store(b1, 1)
            return carry

        lax.fori_loop(0, NB // 2, pair, 0)

    return body(ids2d, table)


TBLK = 2048
NTB = (V_GENE + TBLK - 1) // TBLK
V_PAD = NTB * TBLK


def _tc_transpose_body(in_ref, out_ref):
    y = in_ref[...].T
    out_ref[...] = jnp.concatenate([y[:TBLK // 2], y[TBLK // 2:]], axis=1)


def _tc_transpose(table_t):
    return pl.pallas_call(
        _tc_transpose_body,
        grid=(NTB,),
        in_specs=[pl.BlockSpec((D, TBLK), lambda i: (0, i))],
        out_specs=pl.BlockSpec((TBLK // 2, 128), lambda i: (i, 0)),
        out_shape=jax.ShapeDtypeStruct((V_PAD // 2, 128), jnp.float32),
    )(table_t)


def _tc_combine_body(mod_ref, expt_ref, gene_ref, emodt_ref, w_ref, out_ref):
    gene = gene_ref[...]
    for h in range(2):
        mod = mod_ref[h, 0, 0, :]
        oht = (lax.broadcasted_iota(jnp.int32, (V_MOD_PAD, BT), 0)
               == mod[None, :]).astype(jnp.float32)
        acc = jnp.dot(emodt_ref[...], oht, preferred_element_type=jnp.float32)
        acc += jnp.dot(w_ref[...], expt_ref[h, 0],
                       preferred_element_type=jnp.float32)
        out_ref[h, 0] = acc + gene[:, h * D:(h + 1) * D].T


def _tc_combine(mod4d, expt4d, gene2d, emodt_pad, w):
    grid = (N // 2,)
    return pl.pallas_call(
        _tc_combine_body,
        grid=grid,
        in_specs=[
            pl.BlockSpec((2, 1, 1, BT), lambda i: (0, i, 0, 0)),
            pl.BlockSpec((2, 1, V_EXPR, BT), lambda i: (0, i, 0, 0)),
            pl.BlockSpec((BT, 128), lambda i: (i, 0)),
            pl.BlockSpec((D, V_MOD_PAD), lambda i: (0, 0)),
            pl.BlockSpec((D, V_EXPR), lambda i: (0, 0)),
        ],
        out_specs=pl.BlockSpec((2, 1, D, BT), lambda i: (0, i, 0, 0)),
        out_shape=jax.ShapeDtypeStruct((2, N // 2, D, C), jnp.float32),
    )(mod4d, expt4d, gene2d, emodt_pad, w)


def kernel(gene_id, modality, expression, E_gene, E_modality, W_expr):
    ids2d = gene_id.reshape(T // CB, CB)
    table_rm = _tc_transpose(E_gene.T)
    table = table_rm.reshape(V_PAD * D).reshape(V_PAD, D)
    gathered = _sc_gather(ids2d, table)
    mod4d = modality.reshape(2, N // 2, 1, C)
    expt4d = jnp.transpose(expression, (0, 2, 1)).reshape(2, N // 2, V_EXPR, C)
    emodt_pad = jnp.zeros((D, V_MOD_PAD), jnp.float32).at[:, :V_MOD].set(
        E_modality.T)
    out = _tc_combine(mod4d, expt4d, gathered, emodt_pad, W_expr)
    return jnp.transpose(out.reshape(N, D, C), (0, 2, 1))
